# initial kernel scaffold (unmeasured)
import jax
import jax.numpy as jnp
from jax import lax
from jax.experimental import pallas as pl
from jax.experimental.pallas import tpu as pltpu


def kernel(
    x,
):
    def body(*refs):
        pass

    out_shape = jax.ShapeDtypeStruct(..., jnp.float32)
    return pl.pallas_call(body, out_shape=out_shape)(...)


# baseline (device time: 12615 ns/iter reference)
import jax
import jax.numpy as jnp
from jax import lax
from jax.experimental import pallas as pl
from jax.experimental.pallas import tpu as pltpu


def kernel(x):
    m, n = x.shape

    def body(x_ref, out_ref, send_ref, recv1_ref, recv2_ref, send_sems, recv_sems):
        my_x = lax.axis_index("x")
        my_y = lax.axis_index("y")
        my_z = lax.axis_index("z")
        p1 = (my_x, my_y, my_z ^ 1)
        p2 = (my_x, my_y, my_z ^ 2)

        barrier = pltpu.get_barrier_semaphore()
        for p in (p1, p2):
            pl.semaphore_signal(
                barrier, inc=1, device_id=p, device_id_type=pl.DeviceIdType.MESH
            )
        pl.semaphore_wait(barrier, 2)

        send_ref[...] = x_ref[...].astype(jnp.bfloat16)
        rdma1 = pltpu.make_async_remote_copy(
            src_ref=send_ref,
            dst_ref=recv1_ref,
            send_sem=send_sems.at[0],
            recv_sem=recv_sems.at[0],
            device_id=p1,
            device_id_type=pl.DeviceIdType.MESH,
        )
        rdma1.start()
        rdma1.wait()

        send_ref[...] = send_ref[...] + recv1_ref[...]
        rdma2 = pltpu.make_async_remote_copy(
            src_ref=send_ref,
            dst_ref=recv2_ref,
            send_sem=send_sems.at[1],
            recv_sem=recv_sems.at[1],
            device_id=p2,
            device_id_type=pl.DeviceIdType.MESH,
        )
        rdma2.start()
        rdma2.wait()

        out_ref[...] = send_ref[...].astype(jnp.float32) + recv2_ref[...].astype(
            jnp.float32
        )

    return pl.pallas_call(
        body,
        out_shape=jax.ShapeDtypeStruct((m, n), jnp.float32),
        in_specs=[pl.BlockSpec(memory_space=pltpu.VMEM)],
        out_specs=pl.BlockSpec(memory_space=pltpu.VMEM),
        scratch_shapes=[
            pltpu.VMEM((m, n), jnp.bfloat16),
            pltpu.VMEM((m, n), jnp.bfloat16),
            pltpu.VMEM((m, n), jnp.bfloat16),
            pltpu.SemaphoreType.DMA((2,)),
            pltpu.SemaphoreType.DMA((2,)),
        ],
        compiler_params=pltpu.CompilerParams(collective_id=0),
    )(x)


# device time: 11671 ns/iter; 1.0809x vs baseline; 1.0809x over previous
import jax
import jax.numpy as jnp
from jax import lax
from jax.experimental import pallas as pl
from jax.experimental.pallas import tpu as pltpu

N_CHUNKS = 4


def kernel(x):
    m, n = x.shape
    rows = m // N_CHUNKS

    def body(
        x_ref,
        out_ref,
        send_ref,
        recv1_ref,
        recv2_ref,
        send_sems1,
        recv_sems1,
        send_sems2,
        recv_sems2,
    ):
        my_x = lax.axis_index("x")
        my_y = lax.axis_index("y")
        my_z = lax.axis_index("z")
        p1 = (my_x, my_y, my_z ^ 1)
        p2 = (my_x, my_y, my_z ^ 2)

        barrier = pltpu.get_barrier_semaphore()
        for p in (p1, p2):
            pl.semaphore_signal(
                barrier, inc=1, device_id=p, device_id_type=pl.DeviceIdType.MESH
            )
        send_ref[...] = x_ref[...].astype(jnp.bfloat16)
        pl.semaphore_wait(barrier, 2)

        def chunk(ref, c):
            return ref.at[pl.ds(c * rows, rows), :]

        rdma1 = []
        for c in range(N_CHUNKS):
            r = pltpu.make_async_remote_copy(
                src_ref=chunk(send_ref, c),
                dst_ref=chunk(recv1_ref, c),
                send_sem=send_sems1.at[c],
                recv_sem=recv_sems1.at[c],
                device_id=p1,
                device_id_type=pl.DeviceIdType.MESH,
            )
            r.start()
            rdma1.append(r)

        rdma2 = []
        for c in range(N_CHUNKS):
            rdma1[c].wait()
            chunk(send_ref, c)[...] = chunk(send_ref, c)[...] + chunk(recv1_ref, c)[...]
            r = pltpu.make_async_remote_copy(
                src_ref=chunk(send_ref, c),
                dst_ref=chunk(recv2_ref, c),
                send_sem=send_sems2.at[c],
                recv_sem=recv_sems2.at[c],
                device_id=p2,
                device_id_type=pl.DeviceIdType.MESH,
            )
            r.start()
            rdma2.append(r)

        for c in range(N_CHUNKS):
            rdma2[c].wait()
            chunk(out_ref, c)[...] = chunk(send_ref, c)[...].astype(
                jnp.float32
            ) + chunk(recv2_ref, c)[...].astype(jnp.float32)

    return pl.pallas_call(
        body,
        out_shape=jax.ShapeDtypeStruct((m, n), jnp.float32),
        in_specs=[pl.BlockSpec(memory_space=pltpu.VMEM)],
        out_specs=pl.BlockSpec(memory_space=pltpu.VMEM),
        scratch_shapes=[
            pltpu.VMEM((m, n), jnp.bfloat16),
            pltpu.VMEM((m, n), jnp.bfloat16),
            pltpu.VMEM((m, n), jnp.bfloat16),
            pltpu.SemaphoreType.DMA((N_CHUNKS,)),
            pltpu.SemaphoreType.DMA((N_CHUNKS,)),
            pltpu.SemaphoreType.DMA((N_CHUNKS,)),
            pltpu.SemaphoreType.DMA((N_CHUNKS,)),
        ],
        compiler_params=pltpu.CompilerParams(collective_id=0),
    )(x)
